# Initial kernel scaffold; baseline (speedup 1.0000x reference)
import functools

import jax
import jax.numpy as jnp
from jax import lax
from jax.experimental import pallas as pl
from jax.experimental.pallas import tpu as pltpu
from jax.experimental.pallas import tpu_sc as plsc

_EPS = 1e-5
_SCALING = 0.1

_DIM = 64
_R = 16

_NC = 2
_NS = 16
_NW = _NC * _NS

_STEP = 128


def _sc_gather_kernel(n_per_w, n_steps, idx_hbm, base_hbm, lora_hbm,
                      out_base_hbm, out_lora_hbm,
                      idx_v, brows, lrows, sem_i, sem_b, sem_l):
  wid = lax.axis_index("s") * _NC + lax.axis_index("c")
  gbase = wid * n_per_w

  cp = pltpu.async_copy(idx_hbm.at[pl.ds(gbase, n_per_w)], idx_v, sem_i)
  cp.wait()

  def step(j, carry):
    off = j * _STEP
    cb = pltpu.async_copy(base_hbm.at[idx_v.at[pl.ds(off, _STEP)]], brows,
                          sem_b)
    cl = pltpu.async_copy(lora_hbm.at[idx_v.at[pl.ds(off, _STEP)]], lrows,
                          sem_l)
    cb.wait()
    cl.wait()
    pltpu.sync_copy(brows, out_base_hbm.at[pl.ds(gbase + off, _STEP)])
    pltpu.sync_copy(lrows, out_lora_hbm.at[pl.ds(gbase + off, _STEP)])
    return carry

  lax.fori_loop(0, n_steps, step, 0)


def _sc_gather(idx, base_weight, lora_B):
  n = idx.shape[0]
  n_per_w = n // _NW
  n_steps = n_per_w // _STEP
  mesh = plsc.VectorSubcoreMesh(core_axis_name="c", subcore_axis_name="s")
  return pl.kernel(
      functools.partial(_sc_gather_kernel, n_per_w, n_steps),
      out_type=[
          jax.ShapeDtypeStruct((n, _DIM), jnp.float32),
          jax.ShapeDtypeStruct((n, _R), jnp.float32),
      ],
      mesh=mesh,
      scratch_types=[
          pltpu.VMEM((n_per_w,), jnp.int32),
          pltpu.VMEM((_STEP, _DIM), jnp.float32),
          pltpu.VMEM((_STEP, _R), jnp.float32),
          pltpu.SemaphoreType.DMA,
          pltpu.SemaphoreType.DMA,
          pltpu.SemaphoreType.DMA,
      ],
  )(idx, base_weight, lora_B)


def _tc_math_kernel(base_ref, lora_ref, a_ref, out_ref):
  x = base_ref[...]
  b = lora_ref[...]
  A = a_ref[...]

  max_norm = 1.0 - _EPS

  xn = jnp.sqrt(jnp.sum(x * x, axis=-1, keepdims=True))
  scale = jnp.where(xn > max_norm, max_norm / jnp.maximum(xn, 1e-15), 1.0)
  x = x * scale

  v = _SCALING * jnp.dot(b, A, preferred_element_type=jnp.float32)
  n = jnp.maximum(jnp.sqrt(jnp.sum(v * v, axis=-1, keepdims=True)), 1e-15)
  y = jnp.tanh(n) * v / n

  xy = jnp.sum(x * y, axis=-1, keepdims=True)
  x2 = jnp.sum(x * x, axis=-1, keepdims=True)
  y2 = jnp.sum(y * y, axis=-1, keepdims=True)
  num = (1.0 + 2.0 * xy + y2) * x + (1.0 - x2) * y
  den = jnp.maximum(1.0 + 2.0 * xy + x2 * y2, 1e-15)
  r = num / den

  rn = jnp.sqrt(jnp.sum(r * r, axis=-1, keepdims=True))
  rscale = jnp.where(rn > max_norm, max_norm / jnp.maximum(rn, 1e-15), 1.0)
  out_ref[...] = r * rscale


def _tc_math(gbase, glora, lora_A, blk=4096):
  n = gbase.shape[0]
  grid = (n // blk,)
  return pl.pallas_call(
      _tc_math_kernel,
      grid=grid,
      in_specs=[
          pl.BlockSpec((blk, _DIM), lambda i: (i, 0)),
          pl.BlockSpec((blk, _R), lambda i: (i, 0)),
          pl.BlockSpec((_R, _DIM), lambda i: (0, 0)),
      ],
      out_specs=pl.BlockSpec((blk, _DIM), lambda i: (i, 0)),
      out_shape=jax.ShapeDtypeStruct((n, _DIM), jnp.float32),
  )(gbase, glora, lora_A)


def kernel(indices, base_weight, lora_A, lora_B):
  bsz, seq = indices.shape
  idx = indices.reshape(-1).astype(jnp.int32)
  gbase, glora = _sc_gather(idx, base_weight, lora_B)
  out = _tc_math(gbase, glora, lora_A)
  return out.reshape(bsz, seq, _DIM)


# speedup vs baseline: 3.0977x; 3.0977x over previous
import functools

import jax
import jax.numpy as jnp
from jax import lax
from jax.experimental import pallas as pl
from jax.experimental.pallas import tpu as pltpu
from jax.experimental.pallas import tpu_sc as plsc

_EPS = 1e-5
_SCALING = 0.1

_DIM = 64
_R = 16

_NC = 2
_NS = 16
_NW = _NC * _NS

_STEP = 128


def _sc_gather_kernel(n_per_w, n_steps, idx_hbm, base_hbm, lora_hbm,
                      out_base_hbm, out_lora_hbm,
                      idx_v, brows, lrows, sem_i, sem_b, sem_l):
  wid = lax.axis_index("s") * _NC + lax.axis_index("c")
  gbase = wid * n_per_w

  cp = pltpu.async_copy(idx_hbm.at[pl.ds(gbase, n_per_w)], idx_v, sem_i)
  cp.wait()

  def step(j, carry):
    off = j * _STEP
    cb = pltpu.async_copy(base_hbm.at[idx_v.at[pl.ds(off, _STEP)]], brows,
                          sem_b)
    cl = pltpu.async_copy(lora_hbm.at[idx_v.at[pl.ds(off, _STEP)]], lrows,
                          sem_l)
    cb.wait()
    cl.wait()
    pltpu.sync_copy(brows, out_base_hbm.at[pl.ds(gbase + off, _STEP)])
    pltpu.sync_copy(lrows, out_lora_hbm.at[pl.ds(gbase + off, _STEP)])
    return carry

  lax.fori_loop(0, n_steps, step, 0)


def _sc_gather(idx, base_weight, lora_B):
  n = idx.shape[0]
  n_per_w = n // _NW
  n_steps = n_per_w // _STEP
  mesh = plsc.VectorSubcoreMesh(core_axis_name="c", subcore_axis_name="s")
  return pl.kernel(
      functools.partial(_sc_gather_kernel, n_per_w, n_steps),
      out_type=[
          jax.ShapeDtypeStruct((n, _DIM), jnp.float32),
          jax.ShapeDtypeStruct((n, _R), jnp.float32),
      ],
      mesh=mesh,
      scratch_types=[
          pltpu.VMEM((n_per_w,), jnp.int32),
          pltpu.VMEM((_STEP, _DIM), jnp.float32),
          pltpu.VMEM((_STEP, _R), jnp.float32),
          pltpu.SemaphoreType.DMA,
          pltpu.SemaphoreType.DMA,
          pltpu.SemaphoreType.DMA,
      ],
      compiler_params=pltpu.CompilerParams(use_tc_tiling_on_sc=False),
  )(idx, base_weight, lora_B)


def _tc_math_kernel(base_ref, lora_ref, a_ref, out_ref):
  x = base_ref[...]
  b = lora_ref[...]
  A = a_ref[...]

  max_norm = 1.0 - _EPS

  xn = jnp.sqrt(jnp.sum(x * x, axis=-1, keepdims=True))
  scale = jnp.where(xn > max_norm, max_norm / jnp.maximum(xn, 1e-15), 1.0)
  x = x * scale

  v = _SCALING * jnp.dot(b, A, preferred_element_type=jnp.float32)
  n = jnp.maximum(jnp.sqrt(jnp.sum(v * v, axis=-1, keepdims=True)), 1e-15)
  y = jnp.tanh(n) * v / n

  xy = jnp.sum(x * y, axis=-1, keepdims=True)
  x2 = jnp.sum(x * x, axis=-1, keepdims=True)
  y2 = jnp.sum(y * y, axis=-1, keepdims=True)
  num = (1.0 + 2.0 * xy + y2) * x + (1.0 - x2) * y
  den = jnp.maximum(1.0 + 2.0 * xy + x2 * y2, 1e-15)
  r = num / den

  rn = jnp.sqrt(jnp.sum(r * r, axis=-1, keepdims=True))
  rscale = jnp.where(rn > max_norm, max_norm / jnp.maximum(rn, 1e-15), 1.0)
  out_ref[...] = r * rscale


def _tc_math(gbase, glora, lora_A, blk=4096):
  n = gbase.shape[0]
  grid = (n // blk,)
  return pl.pallas_call(
      _tc_math_kernel,
      grid=grid,
      in_specs=[
          pl.BlockSpec((blk, _DIM), lambda i: (i, 0)),
          pl.BlockSpec((blk, _R), lambda i: (i, 0)),
          pl.BlockSpec((_R, _DIM), lambda i: (0, 0)),
      ],
      out_specs=pl.BlockSpec((blk, _DIM), lambda i: (i, 0)),
      out_shape=jax.ShapeDtypeStruct((n, _DIM), jnp.float32),
  )(gbase, glora, lora_A)


def kernel(indices, base_weight, lora_A, lora_B):
  bsz, seq = indices.shape
  idx = indices.reshape(-1).astype(jnp.int32)
  gbase, glora = _sc_gather(idx, base_weight, lora_B)
  out = _tc_math(gbase, glora, lora_A)
  return out.reshape(bsz, seq, _DIM)


# speedup vs baseline: 3.1670x; 1.0224x over previous
import functools

import jax
import jax.numpy as jnp
from jax import lax
from jax.experimental import pallas as pl
from jax.experimental.pallas import tpu as pltpu
from jax.experimental.pallas import tpu_sc as plsc

_EPS = 1e-5
_SCALING = 0.1

_DIM = 64
_R = 16

_NC = 2
_NS = 16
_NW = _NC * _NS

_STEP = 128


_DEPTH = 5


def _sc_gather_kernel(n_per_w, n_groups, idx_hbm, base_hbm, lora_hbm,
                      out_base_hbm, out_lora_hbm,
                      idx_v, brows, lrows, sem_i, gsems_b, gsems_l, wsems):
  wid = lax.axis_index("s") * _NC + lax.axis_index("c")
  gbase = wid * n_per_w

  pltpu.async_copy(idx_hbm.at[pl.ds(gbase, n_per_w)], idx_v, sem_i).wait()

  def fire(j, b):
    off = j * _STEP
    pltpu.async_copy(base_hbm.at[idx_v.at[pl.ds(off, _STEP)]], brows.at[b],
                     gsems_b.at[b])
    pltpu.async_copy(lora_hbm.at[idx_v.at[pl.ds(off, _STEP)]], lrows.at[b],
                     gsems_l.at[b])

  def drain_gather(j, b):
    off = j * _STEP
    pltpu.make_async_copy(base_hbm.at[idx_v.at[pl.ds(off, _STEP)]],
                          brows.at[b], gsems_b.at[b]).wait()
    pltpu.make_async_copy(lora_hbm.at[idx_v.at[pl.ds(off, _STEP)]],
                          lrows.at[b], gsems_l.at[b]).wait()

  for b in range(_DEPTH):
    fire(b, b)

  def group(g, carry):
    for b in range(_DEPTH):
      j = g * _DEPTH + b
      off = j * _STEP
      drain_gather(j, b)
      pltpu.async_copy(brows.at[b], out_base_hbm.at[pl.ds(gbase + off, _STEP)],
                       wsems.at[b])
      pltpu.async_copy(lrows.at[b], out_lora_hbm.at[pl.ds(gbase + off, _STEP)],
                       wsems.at[b])
    for b in range(_DEPTH):
      j = g * _DEPTH + b
      off = j * _STEP
      pltpu.make_async_copy(brows.at[b],
                            out_base_hbm.at[pl.ds(gbase + off, _STEP)],
                            wsems.at[b]).wait()
      pltpu.make_async_copy(lrows.at[b],
                            out_lora_hbm.at[pl.ds(gbase + off, _STEP)],
                            wsems.at[b]).wait()

      @pl.when(g + 1 < n_groups)
      def _():
        fire((g + 1) * _DEPTH + b, b)

    return carry

  lax.fori_loop(0, n_groups, group, 0)


def _sc_gather(idx, base_weight, lora_B):
  n = idx.shape[0]
  n_per_w = n // _NW
  n_groups = n_per_w // (_STEP * _DEPTH)
  mesh = plsc.VectorSubcoreMesh(core_axis_name="c", subcore_axis_name="s")
  return pl.kernel(
      functools.partial(_sc_gather_kernel, n_per_w, n_groups),
      out_type=[
          jax.ShapeDtypeStruct((n, _DIM), jnp.float32),
          jax.ShapeDtypeStruct((n, _R), jnp.float32),
      ],
      mesh=mesh,
      scratch_types=[
          pltpu.VMEM((n_per_w,), jnp.int32),
          pltpu.VMEM((_DEPTH, _STEP, _DIM), jnp.float32),
          pltpu.VMEM((_DEPTH, _STEP, _R), jnp.float32),
          pltpu.SemaphoreType.DMA,
          pltpu.SemaphoreType.DMA((_DEPTH,)),
          pltpu.SemaphoreType.DMA((_DEPTH,)),
          pltpu.SemaphoreType.DMA((_DEPTH,)),
      ],
      compiler_params=pltpu.CompilerParams(use_tc_tiling_on_sc=False),
  )(idx, base_weight, lora_B)


def _tc_math_kernel(base_ref, lora_ref, a_ref, out_ref):
  x = base_ref[...]
  b = lora_ref[...]
  A = a_ref[...]

  max_norm = 1.0 - _EPS

  xn = jnp.sqrt(jnp.sum(x * x, axis=-1, keepdims=True))
  scale = jnp.where(xn > max_norm, max_norm / jnp.maximum(xn, 1e-15), 1.0)
  x = x * scale

  v = _SCALING * jnp.dot(b, A, preferred_element_type=jnp.float32)
  n = jnp.maximum(jnp.sqrt(jnp.sum(v * v, axis=-1, keepdims=True)), 1e-15)
  y = jnp.tanh(n) * v / n

  xy = jnp.sum(x * y, axis=-1, keepdims=True)
  x2 = jnp.sum(x * x, axis=-1, keepdims=True)
  y2 = jnp.sum(y * y, axis=-1, keepdims=True)
  num = (1.0 + 2.0 * xy + y2) * x + (1.0 - x2) * y
  den = jnp.maximum(1.0 + 2.0 * xy + x2 * y2, 1e-15)
  r = num / den

  rn = jnp.sqrt(jnp.sum(r * r, axis=-1, keepdims=True))
  rscale = jnp.where(rn > max_norm, max_norm / jnp.maximum(rn, 1e-15), 1.0)
  out_ref[...] = r * rscale


def _tc_math(gbase, glora, lora_A, blk=4096):
  n = gbase.shape[0]
  grid = (n // blk,)
  return pl.pallas_call(
      _tc_math_kernel,
      grid=grid,
      in_specs=[
          pl.BlockSpec((blk, _DIM), lambda i: (i, 0)),
          pl.BlockSpec((blk, _R), lambda i: (i, 0)),
          pl.BlockSpec((_R, _DIM), lambda i: (0, 0)),
      ],
      out_specs=pl.BlockSpec((blk, _DIM), lambda i: (i, 0)),
      out_shape=jax.ShapeDtypeStruct((n, _DIM), jnp.float32),
  )(gbase, glora, lora_A)


def kernel(indices, base_weight, lora_A, lora_B):
  bsz, seq = indices.shape
  idx = indices.reshape(-1).astype(jnp.int32)
  gbase, glora = _sc_gather(idx, base_weight, lora_B)
  out = _tc_math(gbase, glora, lora_A)
  return out.reshape(bsz, seq, _DIM)
